# Initial kernel scaffold; baseline (speedup 1.0000x reference)
import jax
import jax.numpy as jnp
from jax.experimental import pallas as pl


def kernel(samples, map_node_values, n):
    raise NotImplementedError("write your pallas kernel here")


# speedup vs baseline: 1.7506x; 1.7506x over previous
import functools

import jax
import jax.numpy as jnp
from jax import lax
from jax.experimental import pallas as pl
from jax.experimental.pallas import tpu as pltpu

N_NODES = 100000
D = 128
K = 16
ROWS_PER_BLOCK = 1024
PAD_N = 100352
N_BLOCKS = PAD_N // ROWS_PER_BLOCK


def _dist_body(m_ref, s_ref, out_ref):
    i = pl.program_id(0)
    d = m_ref[...] - s_ref[...]
    d2 = jnp.sum(d * d, axis=1)
    rows = i * ROWS_PER_BLOCK + lax.broadcasted_iota(jnp.int32, (ROWS_PER_BLOCK,), 0)
    out_ref[...] = jnp.where(rows < N_NODES, d2, jnp.inf)


def _topk_body(dist_ref, idx_ref, val_ref):
    v = dist_ref[...]
    pos = lax.broadcasted_iota(jnp.int32, (PAD_N,), 0)
    io16 = lax.broadcasted_iota(jnp.int32, (K,), 0)
    out_i = jnp.zeros((K,), jnp.int32)
    out_v = jnp.zeros((K,), jnp.float32)
    big = jnp.int32(2**31 - 1)
    for k in range(K):
        m = jnp.min(v)
        sel = jnp.min(jnp.where(v == m, pos, big))
        out_i = jnp.where(io16 == k, sel, out_i)
        out_v = jnp.where(io16 == k, m, out_v)
        v = jnp.where(pos == sel, jnp.inf, v)
    idx_ref[...] = out_i
    val_ref[...] = jnp.sqrt(out_v)


@jax.jit
def _som_bmu(samples, map_node_values):
    dist = pl.pallas_call(
        _dist_body,
        grid=(N_BLOCKS,),
        in_specs=[
            pl.BlockSpec((ROWS_PER_BLOCK, D), lambda i: (i, 0)),
            pl.BlockSpec((1, D), lambda i: (0, 0)),
        ],
        out_specs=pl.BlockSpec((ROWS_PER_BLOCK,), lambda i: (i,)),
        out_shape=jax.ShapeDtypeStruct((PAD_N,), jnp.float32),
    )(map_node_values, samples)
    idx, val = pl.pallas_call(
        _topk_body,
        out_shape=(
            jax.ShapeDtypeStruct((K,), jnp.int32),
            jax.ShapeDtypeStruct((K,), jnp.float32),
        ),
    )(dist)
    return idx, val


def kernel(samples, map_node_values, n):
    del n
    return _som_bmu(samples, map_node_values)
